# baseline (speedup 1.0000x reference)
import functools

import jax
import jax.numpy as jnp
from jax.experimental import pallas as pl

G = 16
B = 64
E = 8
F32 = jnp.float32


def _stem_body(x3_ref, w_ref, b_ref, rw_ref, rb_ref, h_ref, wgt_ref, cv_ref,
               *, N, S, Lv):
    X3 = x3_ref[G:G + N, :]
    h = jnp.maximum(
        jnp.dot(X3, w_ref[...], preferred_element_type=F32) + b_ref[...], 0.0)
    h_ref[0:G, :] = jnp.zeros((G, 16), F32)
    h_ref[G:G + N, :] = h
    h_ref[G + N:, :] = jnp.zeros((G, 16), F32)

    cc = jax.lax.broadcasted_iota(jnp.int32, (B, N), 1)
    bb = jax.lax.broadcasted_iota(jnp.int32, (B, N), 0)
    sel = ((cc // S == bb) & (cc % S < Lv)).astype(F32)
    pooled = jnp.dot(sel, h, preferred_element_type=F32) * (1.0 / Lv)
    logits = jnp.dot(pooled, rw_ref[...], preferred_element_type=F32) + rb_ref[...]
    m = jnp.max(logits, axis=1, keepdims=True)
    ex = jnp.exp(logits - m)
    p = ex / jnp.sum(ex, axis=1, keepdims=True)

    ei = jax.lax.broadcasted_iota(jnp.int32, (B, E), 1)
    m1 = jnp.max(p, axis=1, keepdims=True)
    i1 = jnp.min(jnp.where(p == m1, ei, E), axis=1, keepdims=True)
    s1 = ei == i1
    p2 = jnp.where(s1, -jnp.inf, p)
    m2 = jnp.max(p2, axis=1, keepdims=True)
    i2 = jnp.min(jnp.where(p2 == m2, ei, E), axis=1, keepdims=True)
    s2 = ei == i2
    ssum = m1 + m2
    wgt_ref[...] = jnp.where(s1, m1 / ssum, 0.0) + jnp.where(s2, m2 / ssum, 0.0)

    mp = jnp.mean(p, axis=0, keepdims=True)
    mu = jnp.mean(mp, axis=1, keepdims=True)
    varp = jnp.sum((mp - mu) ** 2, axis=1, keepdims=True) * (1.0 / (E - 1))
    cv_ref[...] = varp / (mu + 1e-10) ** 2


def _conv_bn_pool(X, acc_cols, w_ref, b_ref, g_ref, be_ref,
                  *, K, pad, S, Lv, N):
    lmod = jax.lax.broadcasted_iota(jnp.int32, (N, 1), 0) % S
    acc = jnp.zeros((N, acc_cols), F32)
    for k in range(K):
        d = k - pad
        Xs = jax.lax.slice(X, (G + d, 0), (G + d + N, X.shape[1]))
        msk = ((lmod + d >= 0) & (lmod + d < Lv)).astype(F32)
        acc = acc + jnp.dot(Xs * msk, w_ref[0, k], preferred_element_type=F32)
    y = acc + b_ref[0]
    vm = (lmod < Lv).astype(F32)
    inv_cnt = 1.0 / float(B * Lv)
    mean = jnp.sum(y * vm, axis=0, keepdims=True) * inv_cnt
    dev = (y - mean) * vm
    var = jnp.sum(dev * dev, axis=0, keepdims=True) * inv_cnt
    yn = g_ref[0] * (y - mean) * jax.lax.rsqrt(var + 1e-5) + be_ref[0]
    r = jnp.maximum(yn, 0.0)
    r3 = r.reshape(N // 2, 2, acc_cols)
    return jnp.max(r3, axis=1)


def _layer_body(x_ref, w_ref, b_ref, g_ref, be_ref, o_ref,
                *, K, pad, S, Lv, N, Cout, shared_x):
    X = x_ref[...] if shared_x else x_ref[0]
    p = _conv_bn_pool(X, Cout, w_ref, b_ref, g_ref, be_ref,
                      K=K, pad=pad, S=S, Lv=Lv, N=N)
    o_ref[0, 0:G, :] = jnp.zeros((G, Cout), F32)
    o_ref[0, G:G + N // 2, :] = p
    o_ref[0, G + N // 2:, :] = jnp.zeros((G, Cout), F32)


def _layer4_body(x_ref, w_ref, b_ref, g_ref, be_ref, wgt_ref, o_ref,
                 *, K, pad, S, Lv, N, Ct):
    e = pl.program_id(0)
    j = pl.program_id(1)
    X = x_ref[0]
    p = _conv_bn_pool(X, Ct, w_ref, b_ref, g_ref, be_ref,
                      K=K, pad=pad, S=S, Lv=Lv, N=N)
    NP = N // 2
    SP = S // 2
    LP = Lv // 2
    rr = jax.lax.broadcasted_iota(jnp.int32, (B, NP), 1)
    bb = jax.lax.broadcasted_iota(jnp.int32, (B, NP), 0)
    sel = ((rr // SP == bb) & (rr % SP < LP)).astype(F32)
    z = jnp.dot(sel, p, preferred_element_type=F32) * (1.0 / LP)
    wi = jax.lax.broadcasted_iota(jnp.int32, (B, E), 1)
    wcol = jnp.sum(jnp.where(wi == e, wgt_ref[...], 0.0), axis=1, keepdims=True)
    contrib = z * wcol

    @pl.when(e == 0)
    def _():
        o_ref[:, pl.dslice(j * Ct, Ct)] = contrib

    @pl.when(e != 0)
    def _():
        o_ref[:, pl.dslice(j * Ct, Ct)] += contrib


def _head_body(z_ref, w1_ref, b1_ref, w2_ref, b2_ref, w3_ref, b3_ref, o_ref):
    h1 = jnp.maximum(
        jnp.dot(z_ref[...], w1_ref[...], preferred_element_type=F32) + b1_ref[...], 0.0)
    h2 = jnp.maximum(
        jnp.dot(h1, w2_ref[...], preferred_element_type=F32) + b2_ref[...], 0.0)
    o_ref[...] = jnp.dot(h2, w3_ref[...], preferred_element_type=F32) + b3_ref[...]


def _layer_call(X, Wt, b, g, be, *, K, pad, S, Lv, N, Cin, Cout, shared_x):
    body = functools.partial(_layer_body, K=K, pad=pad, S=S, Lv=Lv, N=N,
                             Cout=Cout, shared_x=shared_x)
    if shared_x:
        x_spec = pl.BlockSpec((G + N + G, Cin), lambda e: (0, 0))
    else:
        x_spec = pl.BlockSpec((1, G + N + G, Cin), lambda e: (e, 0, 0))
    return pl.pallas_call(
        body,
        grid=(E,),
        in_specs=[
            x_spec,
            pl.BlockSpec((1, K, Cin, Cout), lambda e: (e, 0, 0, 0)),
            pl.BlockSpec((1, 1, Cout), lambda e: (e, 0, 0)),
            pl.BlockSpec((1, 1, Cout), lambda e: (e, 0, 0)),
            pl.BlockSpec((1, 1, Cout), lambda e: (e, 0, 0)),
        ],
        out_specs=pl.BlockSpec((1, G + N // 2 + G, Cout), lambda e: (e, 0, 0)),
        out_shape=jax.ShapeDtypeStruct((E, G + N // 2 + G, Cout), F32),
    )(X, Wt, b.reshape(E, 1, Cout), g.reshape(E, 1, Cout), be.reshape(E, 1, Cout))


def kernel(x, conv1_W, conv1_b, router_W, router_b,
           eW1, eb1, g1, be1, eW2, eb2, g2, be2,
           eW3, eb3, g3, be3, eW4, eb4, g4, be4,
           fc1_W, fc1_b, fc2_W, fc2_b, fc3_W, fc3_b):
    L = x.shape[2]
    S1, S2, S3, S4 = 192, 96, 48, 24
    L1, L2, L3, L4 = 187, 93, 46, 23
    N1, N2, N3, N4 = B * S1, B * S2, B * S3, B * S4

    x2 = x[:, 0, :]
    zc = jnp.zeros((B, 1), F32)
    xm = jnp.concatenate([zc, x2[:, :L - 1]], axis=1)
    xp = jnp.concatenate([x2[:, 1:], zc], axis=1)
    X3 = jnp.stack([xm, x2, xp], axis=-1)
    X3 = jnp.pad(X3, ((0, 0), (0, S1 - L), (0, 0))).reshape(N1, 3)
    X3 = jnp.pad(X3, ((G, G), (0, 0)))

    w3 = jnp.transpose(conv1_W[:, 0, :], (1, 0))
    cb = conv1_b.reshape(1, 16)
    rWt = jnp.transpose(router_W, (1, 0))
    rb = router_b.reshape(1, E)
    Wt1 = jnp.transpose(eW1, (0, 3, 2, 1))
    Wt2 = jnp.transpose(eW2, (0, 3, 2, 1))
    Wt3 = jnp.transpose(eW3, (0, 3, 2, 1))
    Wt4 = jnp.transpose(eW4, (0, 3, 2, 1))

    h_pad, wgt, cv2 = pl.pallas_call(
        functools.partial(_stem_body, N=N1, S=S1, Lv=L1),
        out_shape=[
            jax.ShapeDtypeStruct((G + N1 + G, 16), F32),
            jax.ShapeDtypeStruct((B, E), F32),
            jax.ShapeDtypeStruct((1, 1), F32),
        ],
    )(X3, w3, cb, rWt, rb)

    o1 = _layer_call(h_pad, Wt1, eb1, g1, be1, K=3, pad=1, S=S1, Lv=L1,
                     N=N1, Cin=16, Cout=64, shared_x=True)
    o2 = _layer_call(o1, Wt2, eb2, g2, be2, K=9, pad=4, S=S2, Lv=L2,
                     N=N2, Cin=64, Cout=128, shared_x=False)
    o3 = _layer_call(o2, Wt3, eb3, g3, be3, K=11, pad=5, S=S3, Lv=L3,
                     N=N3, Cin=128, Cout=256, shared_x=False)

    CT = 128
    zsum = pl.pallas_call(
        functools.partial(_layer4_body, K=25, pad=12, S=S4, Lv=L4, N=N4, Ct=CT),
        grid=(E, 512 // CT),
        in_specs=[
            pl.BlockSpec((1, G + N4 + G, 256), lambda e, j: (e, 0, 0)),
            pl.BlockSpec((1, 25, 256, CT), lambda e, j: (e, 0, 0, j)),
            pl.BlockSpec((1, 1, CT), lambda e, j: (e, 0, j)),
            pl.BlockSpec((1, 1, CT), lambda e, j: (e, 0, j)),
            pl.BlockSpec((1, 1, CT), lambda e, j: (e, 0, j)),
            pl.BlockSpec((B, E), lambda e, j: (0, 0)),
        ],
        out_specs=pl.BlockSpec((B, 512), lambda e, j: (0, 0)),
        out_shape=jax.ShapeDtypeStruct((B, 512), F32),
    )(o3, Wt4, eb4.reshape(E, 1, 512), g4.reshape(E, 1, 512),
      be4.reshape(E, 1, 512), wgt)

    z = pl.pallas_call(
        _head_body,
        out_shape=jax.ShapeDtypeStruct((B, 5), F32),
    )(zsum, jnp.transpose(fc1_W, (1, 0)), fc1_b.reshape(1, -1),
      jnp.transpose(fc2_W, (1, 0)), fc2_b.reshape(1, -1),
      jnp.transpose(fc3_W, (1, 0)), fc3_b.reshape(1, -1))

    return (z, cv2[0, 0])


# speedup vs baseline: 1.3166x; 1.3166x over previous
import functools

import jax
import jax.numpy as jnp
from jax.experimental import pallas as pl

G = 16
B = 64
E = 8
F32 = jnp.float32


def _stem_body(x3_ref, w_ref, b_ref, rw_ref, rb_ref, h_ref, wgt_ref, cv_ref,
               *, N, S, Lv):
    X3 = x3_ref[G:G + N, :]
    h = jnp.maximum(
        jnp.dot(X3, w_ref[...], preferred_element_type=F32) + b_ref[...], 0.0)
    lmod = jax.lax.broadcasted_iota(jnp.int32, (N, 16), 0) % S
    h = jnp.where(lmod < Lv, h, 0.0)
    h_ref[0:G, :] = jnp.zeros((G, 16), F32)
    h_ref[G:G + N, :] = h
    h_ref[G + N:, :] = jnp.zeros((G, 16), F32)

    cc = jax.lax.broadcasted_iota(jnp.int32, (B, N), 1)
    bb = jax.lax.broadcasted_iota(jnp.int32, (B, N), 0)
    sel = ((cc // S == bb) & (cc % S < Lv)).astype(F32)
    pooled = jnp.dot(sel, h, preferred_element_type=F32) * (1.0 / Lv)
    logits = jnp.dot(pooled, rw_ref[...], preferred_element_type=F32) + rb_ref[...]
    m = jnp.max(logits, axis=1, keepdims=True)
    ex = jnp.exp(logits - m)
    p = ex / jnp.sum(ex, axis=1, keepdims=True)

    ei = jax.lax.broadcasted_iota(jnp.int32, (B, E), 1)
    m1 = jnp.max(p, axis=1, keepdims=True)
    i1 = jnp.min(jnp.where(p == m1, ei, E), axis=1, keepdims=True)
    s1 = ei == i1
    p2 = jnp.where(s1, -jnp.inf, p)
    m2 = jnp.max(p2, axis=1, keepdims=True)
    i2 = jnp.min(jnp.where(p2 == m2, ei, E), axis=1, keepdims=True)
    s2 = ei == i2
    ssum = m1 + m2
    wgt_ref[...] = jnp.where(s1, m1 / ssum, 0.0) + jnp.where(s2, m2 / ssum, 0.0)

    mp = jnp.mean(p, axis=0, keepdims=True)
    mu = jnp.mean(mp, axis=1, keepdims=True)
    varp = jnp.sum((mp - mu) ** 2, axis=1, keepdims=True) * (1.0 / (E - 1))
    cv_ref[...] = varp / (mu + 1e-10) ** 2


def _conv_bn_pool(X, acc_cols, w_ref, b_ref, g_ref, be_ref,
                  *, K, pad, S, Lv, N, zero_out):
    Cin = X.shape[1]
    slack = S - Lv
    lmodU = (jax.lax.broadcasted_iota(jnp.int32, (N, Cin), 0) % S
             ).astype(jnp.uint32)
    acc = None
    for k in range(K):
        d = k - pad
        Xs = jax.lax.slice(X, (G + d, 0), (G + d + N, Cin))
        if abs(d) > slack:
            cond = lmodU + jnp.uint32(d % (1 << 32)) < jnp.uint32(Lv)
            Xs = jnp.where(cond, Xs, 0.0)
        t = jnp.dot(Xs, w_ref[0, k], preferred_element_type=F32)
        acc = t if acc is None else acc + t
    y = acc + b_ref[0]
    lmodO = jax.lax.broadcasted_iota(jnp.int32, (N, acc_cols), 0) % S
    wy = jnp.where(lmodO < Lv, y, 0.0)
    inv_cnt = 1.0 / float(B * Lv)
    mean = jnp.sum(wy, axis=0, keepdims=True) * inv_cnt
    ey2 = jnp.sum(wy * y, axis=0, keepdims=True) * inv_cnt
    var = ey2 - mean * mean
    sc = g_ref[0] * jax.lax.rsqrt(var + 1e-5)
    sh = be_ref[0] - mean * sc
    r = jnp.maximum(y * sc + sh, 0.0)
    r3 = r.reshape(N // 2, 2, acc_cols)
    p = jnp.max(r3, axis=1)
    if zero_out:
        lmodP = jax.lax.broadcasted_iota(jnp.int32, (N // 2, acc_cols), 0) % (S // 2)
        p = jnp.where(lmodP < Lv // 2, p, 0.0)
    return p


def _layer_body(x_ref, w_ref, b_ref, g_ref, be_ref, o_ref,
                *, K, pad, S, Lv, N, Cout, shared_x):
    X = x_ref[...] if shared_x else x_ref[0]
    p = _conv_bn_pool(X, Cout, w_ref, b_ref, g_ref, be_ref,
                      K=K, pad=pad, S=S, Lv=Lv, N=N, zero_out=True)
    o_ref[0, 0:G, :] = jnp.zeros((G, Cout), F32)
    o_ref[0, G:G + N // 2, :] = p
    o_ref[0, G + N // 2:, :] = jnp.zeros((G, Cout), F32)


def _layer4_body(x_ref, w_ref, b_ref, g_ref, be_ref, wgt_ref, o_ref,
                 *, K, pad, S, Lv, N, Ct):
    e = pl.program_id(0)
    j = pl.program_id(1)
    X = x_ref[0]
    p = _conv_bn_pool(X, Ct, w_ref, b_ref, g_ref, be_ref,
                      K=K, pad=pad, S=S, Lv=Lv, N=N, zero_out=False)
    NP = N // 2
    SP = S // 2
    LP = Lv // 2
    rr = jax.lax.broadcasted_iota(jnp.int32, (B, NP), 1)
    bb = jax.lax.broadcasted_iota(jnp.int32, (B, NP), 0)
    sel = ((rr // SP == bb) & (rr % SP < LP)).astype(F32)
    z = jnp.dot(sel, p, preferred_element_type=F32) * (1.0 / LP)
    wi = jax.lax.broadcasted_iota(jnp.int32, (B, E), 1)
    wcol = jnp.sum(jnp.where(wi == e, wgt_ref[...], 0.0), axis=1, keepdims=True)
    contrib = z * wcol

    @pl.when(e == 0)
    def _():
        o_ref[:, pl.dslice(j * Ct, Ct)] = contrib

    @pl.when(e != 0)
    def _():
        o_ref[:, pl.dslice(j * Ct, Ct)] += contrib


def _head_body(z_ref, w1_ref, b1_ref, w2_ref, b2_ref, w3_ref, b3_ref, o_ref):
    h1 = jnp.maximum(
        jnp.dot(z_ref[...], w1_ref[...], preferred_element_type=F32) + b1_ref[...], 0.0)
    h2 = jnp.maximum(
        jnp.dot(h1, w2_ref[...], preferred_element_type=F32) + b2_ref[...], 0.0)
    o_ref[...] = jnp.dot(h2, w3_ref[...], preferred_element_type=F32) + b3_ref[...]


def _layer_call(X, Wt, b, g, be, *, K, pad, S, Lv, N, Cin, Cout, shared_x):
    body = functools.partial(_layer_body, K=K, pad=pad, S=S, Lv=Lv, N=N,
                             Cout=Cout, shared_x=shared_x)
    if shared_x:
        x_spec = pl.BlockSpec((G + N + G, Cin), lambda e: (0, 0))
    else:
        x_spec = pl.BlockSpec((1, G + N + G, Cin), lambda e: (e, 0, 0))
    return pl.pallas_call(
        body,
        grid=(E,),
        in_specs=[
            x_spec,
            pl.BlockSpec((1, K, Cin, Cout), lambda e: (e, 0, 0, 0)),
            pl.BlockSpec((1, 1, Cout), lambda e: (e, 0, 0)),
            pl.BlockSpec((1, 1, Cout), lambda e: (e, 0, 0)),
            pl.BlockSpec((1, 1, Cout), lambda e: (e, 0, 0)),
        ],
        out_specs=pl.BlockSpec((1, G + N // 2 + G, Cout), lambda e: (e, 0, 0)),
        out_shape=jax.ShapeDtypeStruct((E, G + N // 2 + G, Cout), F32),
    )(X, Wt, b.reshape(E, 1, Cout), g.reshape(E, 1, Cout), be.reshape(E, 1, Cout))


def kernel(x, conv1_W, conv1_b, router_W, router_b,
           eW1, eb1, g1, be1, eW2, eb2, g2, be2,
           eW3, eb3, g3, be3, eW4, eb4, g4, be4,
           fc1_W, fc1_b, fc2_W, fc2_b, fc3_W, fc3_b):
    L = x.shape[2]
    S1, S2, S3, S4 = 192, 96, 48, 24
    L1, L2, L3, L4 = 187, 93, 46, 23
    N1, N2, N3, N4 = B * S1, B * S2, B * S3, B * S4

    x2 = x[:, 0, :]
    zc = jnp.zeros((B, 1), F32)
    xm = jnp.concatenate([zc, x2[:, :L - 1]], axis=1)
    xp = jnp.concatenate([x2[:, 1:], zc], axis=1)
    X3 = jnp.stack([xm, x2, xp], axis=-1)
    X3 = jnp.pad(X3, ((0, 0), (0, S1 - L), (0, 0))).reshape(N1, 3)
    X3 = jnp.pad(X3, ((G, G), (0, 0)))

    w3 = jnp.transpose(conv1_W[:, 0, :], (1, 0))
    cb = conv1_b.reshape(1, 16)
    rWt = jnp.transpose(router_W, (1, 0))
    rb = router_b.reshape(1, E)
    Wt1 = jnp.transpose(eW1, (0, 3, 2, 1))
    Wt2 = jnp.transpose(eW2, (0, 3, 2, 1))
    Wt3 = jnp.transpose(eW3, (0, 3, 2, 1))
    Wt4 = jnp.transpose(eW4, (0, 3, 2, 1))

    h_pad, wgt, cv2 = pl.pallas_call(
        functools.partial(_stem_body, N=N1, S=S1, Lv=L1),
        out_shape=[
            jax.ShapeDtypeStruct((G + N1 + G, 16), F32),
            jax.ShapeDtypeStruct((B, E), F32),
            jax.ShapeDtypeStruct((1, 1), F32),
        ],
    )(X3, w3, cb, rWt, rb)

    o1 = _layer_call(h_pad, Wt1, eb1, g1, be1, K=3, pad=1, S=S1, Lv=L1,
                     N=N1, Cin=16, Cout=64, shared_x=True)
    o2 = _layer_call(o1, Wt2, eb2, g2, be2, K=9, pad=4, S=S2, Lv=L2,
                     N=N2, Cin=64, Cout=128, shared_x=False)
    o3 = _layer_call(o2, Wt3, eb3, g3, be3, K=11, pad=5, S=S3, Lv=L3,
                     N=N3, Cin=128, Cout=256, shared_x=False)

    CT = 128
    zsum = pl.pallas_call(
        functools.partial(_layer4_body, K=25, pad=12, S=S4, Lv=L4, N=N4, Ct=CT),
        grid=(E, 512 // CT),
        in_specs=[
            pl.BlockSpec((1, G + N4 + G, 256), lambda e, j: (e, 0, 0)),
            pl.BlockSpec((1, 25, 256, CT), lambda e, j: (e, 0, 0, j)),
            pl.BlockSpec((1, 1, CT), lambda e, j: (e, 0, j)),
            pl.BlockSpec((1, 1, CT), lambda e, j: (e, 0, j)),
            pl.BlockSpec((1, 1, CT), lambda e, j: (e, 0, j)),
            pl.BlockSpec((B, E), lambda e, j: (0, 0)),
        ],
        out_specs=pl.BlockSpec((B, 512), lambda e, j: (0, 0)),
        out_shape=jax.ShapeDtypeStruct((B, 512), F32),
    )(o3, Wt4, eb4.reshape(E, 1, 512), g4.reshape(E, 1, 512),
      be4.reshape(E, 1, 512), wgt)

    z = pl.pallas_call(
        _head_body,
        out_shape=jax.ShapeDtypeStruct((B, 5), F32),
    )(zsum, jnp.transpose(fc1_W, (1, 0)), fc1_b.reshape(1, -1),
      jnp.transpose(fc2_W, (1, 0)), fc2_b.reshape(1, -1),
      jnp.transpose(fc3_W, (1, 0)), fc3_b.reshape(1, -1))

    return (z, cv2[0, 0])


# speedup vs baseline: 1.5130x; 1.1491x over previous
import functools

import jax
import jax.numpy as jnp
from jax.experimental import pallas as pl

G = 16
B = 64
E = 8
F32 = jnp.float32


def _stem_body(x3_ref, w_ref, b_ref, rw_ref, rb_ref, h_ref, wgt_ref, cv_ref,
               *, N, S, Lv):
    X3 = x3_ref[G:G + N, :]
    h = jnp.maximum(
        jnp.dot(X3, w_ref[...], preferred_element_type=F32) + b_ref[...], 0.0)
    lmod = jax.lax.broadcasted_iota(jnp.int32, (N, 16), 0) % S
    h = jnp.where(lmod < Lv, h, 0.0)
    h_ref[0:G, :] = jnp.zeros((G, 16), F32)
    h_ref[G:G + N, :] = h
    h_ref[G + N:, :] = jnp.zeros((G, 16), F32)

    cc = jax.lax.broadcasted_iota(jnp.int32, (B, N), 1)
    bb = jax.lax.broadcasted_iota(jnp.int32, (B, N), 0)
    sel = ((cc // S == bb) & (cc % S < Lv)).astype(F32)
    pooled = jnp.dot(sel, h, preferred_element_type=F32) * (1.0 / Lv)
    logits = jnp.dot(pooled, rw_ref[...], preferred_element_type=F32) + rb_ref[...]
    m = jnp.max(logits, axis=1, keepdims=True)
    ex = jnp.exp(logits - m)
    p = ex / jnp.sum(ex, axis=1, keepdims=True)

    ei = jax.lax.broadcasted_iota(jnp.int32, (B, E), 1)
    m1 = jnp.max(p, axis=1, keepdims=True)
    i1 = jnp.min(jnp.where(p == m1, ei, E), axis=1, keepdims=True)
    s1 = ei == i1
    p2 = jnp.where(s1, -jnp.inf, p)
    m2 = jnp.max(p2, axis=1, keepdims=True)
    i2 = jnp.min(jnp.where(p2 == m2, ei, E), axis=1, keepdims=True)
    s2 = ei == i2
    ssum = m1 + m2
    wgt_ref[...] = jnp.where(s1, m1 / ssum, 0.0) + jnp.where(s2, m2 / ssum, 0.0)

    mp = jnp.mean(p, axis=0, keepdims=True)
    mu = jnp.mean(mp, axis=1, keepdims=True)
    varp = jnp.sum((mp - mu) ** 2, axis=1, keepdims=True) * (1.0 / (E - 1))
    cv_ref[...] = varp / (mu + 1e-10) ** 2


def _conv_bn_pool(X, acc_cols, w_ref, b_ref, g_ref, be_ref,
                  *, K, pad, S, Lv, N, zero_out):
    Cin = X.shape[1]
    slack = S - Lv
    lmodU = (jax.lax.broadcasted_iota(jnp.int32, (N, Cin), 0) % S
             ).astype(jnp.uint32)
    acc = None
    for k in range(K):
        d = k - pad
        Xs = jax.lax.slice(X, (G + d, 0), (G + d + N, Cin))
        if abs(d) > slack:
            cond = lmodU + jnp.uint32(d % (1 << 32)) < jnp.uint32(Lv)
            Xs = jnp.where(cond, Xs, 0.0)
        t = jnp.dot(Xs, w_ref[0, k], preferred_element_type=F32)
        acc = t if acc is None else acc + t
    y = acc + b_ref[0]
    lmodO = jax.lax.broadcasted_iota(jnp.int32, (N, acc_cols), 0) % S
    wy = jnp.where(lmodO < Lv, y, 0.0)
    inv_cnt = 1.0 / float(B * Lv)
    mean = jnp.sum(wy, axis=0, keepdims=True) * inv_cnt
    ey2 = jnp.sum(wy * y, axis=0, keepdims=True) * inv_cnt
    var = ey2 - mean * mean
    sc = g_ref[0] * jax.lax.rsqrt(var + 1e-5)
    sh = be_ref[0] - mean * sc
    r = jnp.maximum(y * sc + sh, 0.0)
    r3 = r.reshape(N // 2, 2, acc_cols)
    p = jnp.max(r3, axis=1)
    if zero_out:
        lmodP = jax.lax.broadcasted_iota(jnp.int32, (N // 2, acc_cols), 0) % (S // 2)
        p = jnp.where(lmodP < Lv // 2, p, 0.0)
    return p


def _layer_body(x_ref, w_ref, b_ref, g_ref, be_ref, o_ref,
                *, K, pad, S, Lv, N, Cout, shared_x):
    X = x_ref[...] if shared_x else x_ref[0]
    p = _conv_bn_pool(X, Cout, w_ref, b_ref, g_ref, be_ref,
                      K=K, pad=pad, S=S, Lv=Lv, N=N, zero_out=True)
    o_ref[0, 0:G, :] = jnp.zeros((G, Cout), F32)
    o_ref[0, G:G + N // 2, :] = p
    o_ref[0, G + N // 2:, :] = jnp.zeros((G, Cout), F32)


def _layer4_body(x_ref, w_ref, b_ref, g_ref, be_ref, wgt_ref, o_ref,
                 *, K, pad, S, Lv, N, Ct):
    e = pl.program_id(0)
    j = pl.program_id(1)
    X = x_ref[0]
    p = _conv_bn_pool(X, Ct, w_ref, b_ref, g_ref, be_ref,
                      K=K, pad=pad, S=S, Lv=Lv, N=N, zero_out=False)
    NP = N // 2
    SP = S // 2
    LP = Lv // 2
    rr = jax.lax.broadcasted_iota(jnp.int32, (B, NP), 1)
    bb = jax.lax.broadcasted_iota(jnp.int32, (B, NP), 0)
    sel = ((rr // SP == bb) & (rr % SP < LP)).astype(F32)
    z = jnp.dot(sel, p, preferred_element_type=F32) * (1.0 / LP)
    wi = jax.lax.broadcasted_iota(jnp.int32, (B, E), 1)
    wcol = jnp.sum(jnp.where(wi == e, wgt_ref[...], 0.0), axis=1, keepdims=True)
    contrib = z * wcol

    @pl.when(e == 0)
    def _():
        o_ref[:, pl.dslice(j * Ct, Ct)] = contrib

    @pl.when(e != 0)
    def _():
        o_ref[:, pl.dslice(j * Ct, Ct)] += contrib


def _head_body(z_ref, w1_ref, b1_ref, w2_ref, b2_ref, w3_ref, b3_ref, o_ref):
    h1 = jnp.maximum(
        jnp.dot(z_ref[...], w1_ref[...], preferred_element_type=F32) + b1_ref[...], 0.0)
    h2 = jnp.maximum(
        jnp.dot(h1, w2_ref[...], preferred_element_type=F32) + b2_ref[...], 0.0)
    o_ref[...] = jnp.dot(h2, w3_ref[...], preferred_element_type=F32) + b3_ref[...]


def _layer_call(X, Wt, b, g, be, *, K, pad, S, Lv, N, Cin, Cout, shared_x):
    body = functools.partial(_layer_body, K=K, pad=pad, S=S, Lv=Lv, N=N,
                             Cout=Cout, shared_x=shared_x)
    if shared_x:
        x_spec = pl.BlockSpec((G + N + G, Cin), lambda e: (0, 0))
    else:
        x_spec = pl.BlockSpec((1, G + N + G, Cin), lambda e: (e, 0, 0))
    return pl.pallas_call(
        body,
        grid=(E,),
        in_specs=[
            x_spec,
            pl.BlockSpec((1, K, Cin, Cout), lambda e: (e, 0, 0, 0)),
            pl.BlockSpec((1, 1, Cout), lambda e: (e, 0, 0)),
            pl.BlockSpec((1, 1, Cout), lambda e: (e, 0, 0)),
            pl.BlockSpec((1, 1, Cout), lambda e: (e, 0, 0)),
        ],
        out_specs=pl.BlockSpec((1, G + N // 2 + G, Cout), lambda e: (e, 0, 0)),
        out_shape=jax.ShapeDtypeStruct((E, G + N // 2 + G, Cout), F32),
    )(X, Wt, b.reshape(E, 1, Cout), g.reshape(E, 1, Cout), be.reshape(E, 1, Cout))


def kernel(x, conv1_W, conv1_b, router_W, router_b,
           eW1, eb1, g1, be1, eW2, eb2, g2, be2,
           eW3, eb3, g3, be3, eW4, eb4, g4, be4,
           fc1_W, fc1_b, fc2_W, fc2_b, fc3_W, fc3_b):
    L = x.shape[2]
    S1, S2, S3, S4 = 192, 96, 48, 24
    L1, L2, L3, L4 = 187, 93, 46, 23
    N1, N2, N3, N4 = B * S1, B * S2, B * S3, B * S4

    x2 = x[:, 0, :]
    zc = jnp.zeros((B, 1), F32)
    xm = jnp.concatenate([zc, x2[:, :L - 1]], axis=1)
    xp = jnp.concatenate([x2[:, 1:], zc], axis=1)
    X3 = jnp.stack([xm, x2, xp], axis=-1)
    X3 = jnp.pad(X3, ((0, 0), (0, S1 - L), (0, 0))).reshape(N1, 3)
    X3 = jnp.pad(X3, ((G, G), (0, 0)))

    w3 = jnp.transpose(conv1_W[:, 0, :], (1, 0))
    cb = conv1_b.reshape(1, 16)
    rWt = jnp.transpose(router_W, (1, 0))
    rb = router_b.reshape(1, E)
    Wt1 = jnp.transpose(eW1, (0, 3, 2, 1))
    Wt2 = jnp.transpose(eW2, (0, 3, 2, 1))
    Wt3 = jnp.transpose(eW3, (0, 3, 2, 1))
    Wt4 = jnp.transpose(eW4, (0, 3, 2, 1))

    h_pad, wgt, cv2 = pl.pallas_call(
        functools.partial(_stem_body, N=N1, S=S1, Lv=L1),
        out_shape=[
            jax.ShapeDtypeStruct((G + N1 + G, 16), F32),
            jax.ShapeDtypeStruct((B, E), F32),
            jax.ShapeDtypeStruct((1, 1), F32),
        ],
    )(X3, w3, cb, rWt, rb)

    o1 = _layer_call(h_pad, Wt1, eb1, g1, be1, K=3, pad=1, S=S1, Lv=L1,
                     N=N1, Cin=16, Cout=64, shared_x=True)
    o2 = _layer_call(o1, Wt2, eb2, g2, be2, K=9, pad=4, S=S2, Lv=L2,
                     N=N2, Cin=64, Cout=128, shared_x=False)
    o3 = _layer_call(o2, Wt3, eb3, g3, be3, K=11, pad=5, S=S3, Lv=L3,
                     N=N3, Cin=128, Cout=256, shared_x=False)

    CT = 256
    zsum = pl.pallas_call(
        functools.partial(_layer4_body, K=25, pad=12, S=S4, Lv=L4, N=N4, Ct=CT),
        grid=(E, 512 // CT),
        in_specs=[
            pl.BlockSpec((1, G + N4 + G, 256), lambda e, j: (e, 0, 0)),
            pl.BlockSpec((1, 25, 256, CT), lambda e, j: (e, 0, 0, j)),
            pl.BlockSpec((1, 1, CT), lambda e, j: (e, 0, j)),
            pl.BlockSpec((1, 1, CT), lambda e, j: (e, 0, j)),
            pl.BlockSpec((1, 1, CT), lambda e, j: (e, 0, j)),
            pl.BlockSpec((B, E), lambda e, j: (0, 0)),
        ],
        out_specs=pl.BlockSpec((B, 512), lambda e, j: (0, 0)),
        out_shape=jax.ShapeDtypeStruct((B, 512), F32),
    )(o3, Wt4, eb4.reshape(E, 1, 512), g4.reshape(E, 1, 512),
      be4.reshape(E, 1, 512), wgt)

    z = pl.pallas_call(
        _head_body,
        out_shape=jax.ShapeDtypeStruct((B, 5), F32),
    )(zsum, jnp.transpose(fc1_W, (1, 0)), fc1_b.reshape(1, -1),
      jnp.transpose(fc2_W, (1, 0)), fc2_b.reshape(1, -1),
      jnp.transpose(fc3_W, (1, 0)), fc3_b.reshape(1, -1))

    return (z, cv2[0, 0])


# speedup vs baseline: 1.5354x; 1.0149x over previous
import functools

import jax
import jax.numpy as jnp
from jax.experimental import pallas as pl

G = 16
B = 64
E = 8
F32 = jnp.float32


def _stem_body(x3_ref, w_ref, b_ref, rw_ref, rb_ref, h_ref, wgt_ref, cv_ref,
               *, N, S, Lv):
    X3 = x3_ref[G:G + N, :]
    h = jnp.maximum(
        jnp.dot(X3, w_ref[...], preferred_element_type=F32) + b_ref[...], 0.0)
    lmod = jax.lax.broadcasted_iota(jnp.int32, (N, 16), 0) % S
    h = jnp.where(lmod < Lv, h, 0.0)
    h_ref[0:G, :] = jnp.zeros((G, 16), F32)
    h_ref[G:G + N, :] = h
    h_ref[G + N:, :] = jnp.zeros((G, 16), F32)

    cc = jax.lax.broadcasted_iota(jnp.int32, (B, N), 1)
    bb = jax.lax.broadcasted_iota(jnp.int32, (B, N), 0)
    sel = ((cc // S == bb) & (cc % S < Lv)).astype(F32)
    pooled = jnp.dot(sel, h, preferred_element_type=F32) * (1.0 / Lv)
    logits = jnp.dot(pooled, rw_ref[...], preferred_element_type=F32) + rb_ref[...]
    m = jnp.max(logits, axis=1, keepdims=True)
    ex = jnp.exp(logits - m)
    p = ex / jnp.sum(ex, axis=1, keepdims=True)

    ei = jax.lax.broadcasted_iota(jnp.int32, (B, E), 1)
    m1 = jnp.max(p, axis=1, keepdims=True)
    i1 = jnp.min(jnp.where(p == m1, ei, E), axis=1, keepdims=True)
    s1 = ei == i1
    p2 = jnp.where(s1, -jnp.inf, p)
    m2 = jnp.max(p2, axis=1, keepdims=True)
    i2 = jnp.min(jnp.where(p2 == m2, ei, E), axis=1, keepdims=True)
    s2 = ei == i2
    ssum = m1 + m2
    wgt_ref[...] = jnp.where(s1, m1 / ssum, 0.0) + jnp.where(s2, m2 / ssum, 0.0)

    mp = jnp.mean(p, axis=0, keepdims=True)
    mu = jnp.mean(mp, axis=1, keepdims=True)
    varp = jnp.sum((mp - mu) ** 2, axis=1, keepdims=True) * (1.0 / (E - 1))
    cv_ref[...] = varp / (mu + 1e-10) ** 2


def _conv_bn_pool(X, acc_cols, w_ref, b_ref, g_ref, be_ref,
                  *, K, pad, S, Lv, N, zero_out):
    Cin = X.shape[1]
    slack = S - Lv
    lmodU = (jax.lax.broadcasted_iota(jnp.int32, (N, Cin), 0) % S
             ).astype(jnp.uint32)
    acc = None
    for k in range(K):
        d = k - pad
        Xs = jax.lax.slice(X, (G + d, 0), (G + d + N, Cin))
        if abs(d) > slack:
            cond = lmodU + jnp.uint32(d % (1 << 32)) < jnp.uint32(Lv)
            Xs = jnp.where(cond, Xs, 0.0)
        t = jnp.dot(Xs, w_ref[0, k], preferred_element_type=F32)
        acc = t if acc is None else acc + t
    y = acc + b_ref[0]
    lmodO = jax.lax.broadcasted_iota(jnp.int32, (N, acc_cols), 0) % S
    wy = jnp.where(lmodO < Lv, y, 0.0)
    inv_cnt = 1.0 / float(B * Lv)
    mean = jnp.sum(wy, axis=0, keepdims=True) * inv_cnt
    ey2 = jnp.sum(wy * y, axis=0, keepdims=True) * inv_cnt
    var = ey2 - mean * mean
    sc = g_ref[0] * jax.lax.rsqrt(var + 1e-5)
    sh = be_ref[0] - mean * sc
    r = jnp.maximum(y * sc + sh, 0.0)
    r3 = r.reshape(N // 2, 2, acc_cols)
    p = jnp.max(r3, axis=1)
    if zero_out:
        lmodP = jax.lax.broadcasted_iota(jnp.int32, (N // 2, acc_cols), 0) % (S // 2)
        p = jnp.where(lmodP < Lv // 2, p, 0.0)
    return p


def _layer_body(x_ref, w_ref, b_ref, g_ref, be_ref, o_ref,
                *, K, pad, S, Lv, N, Cout, shared_x):
    X = x_ref[...] if shared_x else x_ref[0]
    p = _conv_bn_pool(X, Cout, w_ref, b_ref, g_ref, be_ref,
                      K=K, pad=pad, S=S, Lv=Lv, N=N, zero_out=True)
    o_ref[0, 0:G, :] = jnp.zeros((G, Cout), F32)
    o_ref[0, G:G + N // 2, :] = p
    o_ref[0, G + N // 2:, :] = jnp.zeros((G, Cout), F32)


def _layer4_body(x_ref, w_ref, b_ref, g_ref, be_ref, wgt_ref, o_ref,
                 *, K, pad, S, Lv, N, Ct):
    e = pl.program_id(0)
    j = pl.program_id(1)
    X = x_ref[0]
    p = _conv_bn_pool(X, Ct, w_ref, b_ref, g_ref, be_ref,
                      K=K, pad=pad, S=S, Lv=Lv, N=N, zero_out=False)
    NP = N // 2
    SP = S // 2
    LP = Lv // 2
    rr = jax.lax.broadcasted_iota(jnp.int32, (B, NP), 1)
    bb = jax.lax.broadcasted_iota(jnp.int32, (B, NP), 0)
    sel = ((rr // SP == bb) & (rr % SP < LP)).astype(F32)
    z = jnp.dot(sel, p, preferred_element_type=F32) * (1.0 / LP)
    wi = jax.lax.broadcasted_iota(jnp.int32, (B, E), 1)
    wcol = jnp.sum(jnp.where(wi == e, wgt_ref[...], 0.0), axis=1, keepdims=True)
    contrib = z * wcol

    @pl.when(e == 0)
    def _():
        o_ref[:, pl.dslice(j * Ct, Ct)] = contrib

    @pl.when(e != 0)
    def _():
        o_ref[:, pl.dslice(j * Ct, Ct)] += contrib


def _head_body(z_ref, w1_ref, b1_ref, w2_ref, b2_ref, w3_ref, b3_ref, o_ref):
    h1 = jnp.maximum(
        jnp.dot(z_ref[...], w1_ref[...], preferred_element_type=F32) + b1_ref[...], 0.0)
    h2 = jnp.maximum(
        jnp.dot(h1, w2_ref[...], preferred_element_type=F32) + b2_ref[...], 0.0)
    o_ref[...] = jnp.dot(h2, w3_ref[...], preferred_element_type=F32) + b3_ref[...]


def _layer_call(X, Wt, b, g, be, *, K, pad, S, Lv, N, Cin, Cout, shared_x):
    body = functools.partial(_layer_body, K=K, pad=pad, S=S, Lv=Lv, N=N,
                             Cout=Cout, shared_x=shared_x)
    if shared_x:
        x_spec = pl.BlockSpec((G + N + G, Cin), lambda e: (0, 0))
    else:
        x_spec = pl.BlockSpec((1, G + N + G, Cin), lambda e: (e, 0, 0))
    return pl.pallas_call(
        body,
        grid=(E,),
        in_specs=[
            x_spec,
            pl.BlockSpec((1, K, Cin, Cout), lambda e: (e, 0, 0, 0)),
            pl.BlockSpec((1, 1, Cout), lambda e: (e, 0, 0)),
            pl.BlockSpec((1, 1, Cout), lambda e: (e, 0, 0)),
            pl.BlockSpec((1, 1, Cout), lambda e: (e, 0, 0)),
        ],
        out_specs=pl.BlockSpec((1, G + N // 2 + G, Cout), lambda e: (e, 0, 0)),
        out_shape=jax.ShapeDtypeStruct((E, G + N // 2 + G, Cout), F32),
    )(X, Wt, b.reshape(E, 1, Cout), g.reshape(E, 1, Cout), be.reshape(E, 1, Cout))


def kernel(x, conv1_W, conv1_b, router_W, router_b,
           eW1, eb1, g1, be1, eW2, eb2, g2, be2,
           eW3, eb3, g3, be3, eW4, eb4, g4, be4,
           fc1_W, fc1_b, fc2_W, fc2_b, fc3_W, fc3_b):
    L = x.shape[2]
    S1, S2, S3, S4 = 192, 96, 48, 24
    L1, L2, L3, L4 = 187, 93, 46, 23
    N1, N2, N3, N4 = B * S1, B * S2, B * S3, B * S4

    x2 = x[:, 0, :]
    zc = jnp.zeros((B, 1), F32)
    xm = jnp.concatenate([zc, x2[:, :L - 1]], axis=1)
    xp = jnp.concatenate([x2[:, 1:], zc], axis=1)
    X3 = jnp.stack([xm, x2, xp], axis=-1)
    X3 = jnp.pad(X3, ((0, 0), (0, S1 - L), (0, 0))).reshape(N1, 3)
    X3 = jnp.pad(X3, ((G, G), (0, 0)))

    w3 = jnp.transpose(conv1_W[:, 0, :], (1, 0))
    cb = conv1_b.reshape(1, 16)
    rWt = jnp.transpose(router_W, (1, 0))
    rb = router_b.reshape(1, E)
    Wt1 = jnp.transpose(eW1, (0, 3, 2, 1))
    Wt2 = jnp.transpose(eW2, (0, 3, 2, 1))
    Wt3 = jnp.transpose(eW3, (0, 3, 2, 1))
    Wt4 = jnp.transpose(eW4, (0, 3, 2, 1))

    h_pad, wgt, cv2 = pl.pallas_call(
        functools.partial(_stem_body, N=N1, S=S1, Lv=L1),
        out_shape=[
            jax.ShapeDtypeStruct((G + N1 + G, 16), F32),
            jax.ShapeDtypeStruct((B, E), F32),
            jax.ShapeDtypeStruct((1, 1), F32),
        ],
    )(X3, w3, cb, rWt, rb)

    o1 = _layer_call(h_pad, Wt1, eb1, g1, be1, K=3, pad=1, S=S1, Lv=L1,
                     N=N1, Cin=16, Cout=64, shared_x=True)
    o2 = _layer_call(o1, Wt2, eb2, g2, be2, K=9, pad=4, S=S2, Lv=L2,
                     N=N2, Cin=64, Cout=128, shared_x=False)
    o3 = _layer_call(o2, Wt3, eb3, g3, be3, K=11, pad=5, S=S3, Lv=L3,
                     N=N3, Cin=128, Cout=256, shared_x=False)

    CT = 512
    zsum = pl.pallas_call(
        functools.partial(_layer4_body, K=25, pad=12, S=S4, Lv=L4, N=N4, Ct=CT),
        grid=(E, 512 // CT),
        in_specs=[
            pl.BlockSpec((1, G + N4 + G, 256), lambda e, j: (e, 0, 0)),
            pl.BlockSpec((1, 25, 256, CT), lambda e, j: (e, 0, 0, j)),
            pl.BlockSpec((1, 1, CT), lambda e, j: (e, 0, j)),
            pl.BlockSpec((1, 1, CT), lambda e, j: (e, 0, j)),
            pl.BlockSpec((1, 1, CT), lambda e, j: (e, 0, j)),
            pl.BlockSpec((B, E), lambda e, j: (0, 0)),
        ],
        out_specs=pl.BlockSpec((B, 512), lambda e, j: (0, 0)),
        out_shape=jax.ShapeDtypeStruct((B, 512), F32),
    )(o3, Wt4, eb4.reshape(E, 1, 512), g4.reshape(E, 1, 512),
      be4.reshape(E, 1, 512), wgt)

    z = pl.pallas_call(
        _head_body,
        out_shape=jax.ShapeDtypeStruct((B, 5), F32),
    )(zsum, jnp.transpose(fc1_W, (1, 0)), fc1_b.reshape(1, -1),
      jnp.transpose(fc2_W, (1, 0)), fc2_b.reshape(1, -1),
      jnp.transpose(fc3_W, (1, 0)), fc3_b.reshape(1, -1))

    return (z, cv2[0, 0])
